# Initial kernel scaffold; baseline (speedup 1.0000x reference)
import jax
import jax.numpy as jnp
from jax.experimental import pallas as pl


def kernel(x, params):
    raise NotImplementedError("write your pallas kernel here")


# speedup vs baseline: 4.6094x; 4.6094x over previous
import functools
import math

import jax
import jax.numpy as jnp
import numpy as np
from jax.experimental import pallas as pl

HID = 768
INTER = 1024
HEADS = 12
DH = 64
FACTOR = 5


def _sample_consts():
    cpu = jax.local_devices(backend="cpu")[0]
    out = []
    with jax.default_device(cpu):
        key = jax.random.key(42)
        k0, k1 = jax.random.split(key)
        for k, L in ((k0, 2048), (k1, 1024)):
            sample_k = min(FACTOR * int(math.ceil(math.log(L))), L)
            u = min(FACTOR * int(math.ceil(math.log(L))), L)
            idx = np.asarray(jax.random.randint(k, (L, sample_k), 0, L))
            cnt = np.zeros((L, L), np.float32)
            np.add.at(cnt, (np.arange(L)[:, None], idx), 1.0)
            out.append((cnt, u))
    return out


(_CNT0, _U0), (_CNT1, _U1) = _sample_consts()


def _qkv_kernel(x_ref, wq_ref, wk_ref, wv_ref, bq_ref, bk_ref, bv_ref,
                q_ref, kt_ref, v_ref):
    x = x_ref[...]
    q_ref[0] = jnp.dot(x, wq_ref[0]) + bq_ref[0]
    k = jnp.dot(x, wk_ref[0]) + bk_ref[0]
    kt_ref[0] = k.T
    v_ref[0] = jnp.dot(x, wv_ref[0]) + bv_ref[0]


def _m_kernel(q_ref, kt_ref, cnt_ref, m_ref, *, LK):
    s = jnp.dot(q_ref[0], kt_ref[0])
    c = cnt_ref[...]
    mx = jnp.max(jnp.where(c > 0.0, s, -jnp.inf), axis=1)
    sm = jnp.sum(s * c, axis=1)
    m_ref[0, 0, 0] = mx - sm * (1.0 / LK)


def _topk_kernel(m_ref, top_ref, *, U):
    m = m_ref[...]
    L = m.shape[1]
    iota = jax.lax.broadcasted_iota(jnp.int32, m.shape, 1)
    for u in range(U):
        mx = jnp.max(m, axis=1, keepdims=True)
        amax = jnp.min(jnp.where(m == mx, iota, L), axis=1, keepdims=True)
        top_ref[:, u:u + 1] = amax
        m = jnp.where(iota == amax, -jnp.inf, m)


def _tail_kernel(q_ref, kt_ref, v_ref, topc_ref, topr_ref, wo_ref, bo_ref,
                 x_ref, g_ref, b_ref, o_ref, *, U, scale, NH):
    h = pl.program_id(0)

    @pl.when(h == 0)
    def _():
        o_ref[...] = x_ref[...] + bo_ref[...]

    q = q_ref[0]
    kt = kt_ref[0]
    v = v_ref[0]
    L = q.shape[0]
    pt = (jax.lax.broadcasted_iota(jnp.int32, (U, L), 1)
          == topc_ref[0]).astype(jnp.float32)
    ptt = (jax.lax.broadcasted_iota(jnp.int32, (L, U), 0)
           == topr_ref[0]).astype(jnp.float32)
    qr = jnp.dot(pt, q)
    sc = jnp.dot(qr, kt) * scale
    sc = sc - jnp.max(sc, axis=1, keepdims=True)
    e = jnp.exp(sc)
    attn = e * (1.0 / jnp.sum(e, axis=1, keepdims=True))
    upd = jnp.dot(attn, v)
    mv = jnp.mean(v, axis=0, keepdims=True)
    ctx = mv + jnp.dot(ptt, upd - mv)
    o_ref[...] += jnp.dot(ctx, wo_ref[0])

    @pl.when(h == NH - 1)
    def _():
        o = o_ref[...]
        mu = jnp.mean(o, axis=1, keepdims=True)
        var = jnp.mean((o - mu) ** 2, axis=1, keepdims=True)
        o_ref[...] = (o - mu) * jax.lax.rsqrt(var + 1e-5) * g_ref[...] + b_ref[...]


def _ffn_kernel(x_ref, w1_ref, b1_ref, w2_ref, b2_ref, g2_ref, be2_ref,
                gn_ref, bn_ref, o_ref, *, final):
    x = x_ref[...]
    hdn = jnp.maximum(jnp.dot(x, w1_ref[...]) + b1_ref[...], 0.0)
    y = x + jnp.dot(hdn, w2_ref[...]) + b2_ref[...]
    mu = jnp.mean(y, axis=1, keepdims=True)
    var = jnp.mean((y - mu) ** 2, axis=1, keepdims=True)
    y = (y - mu) * jax.lax.rsqrt(var + 1e-5) * g2_ref[...] + be2_ref[...]
    if final:
        mu = jnp.mean(y, axis=1, keepdims=True)
        var = jnp.mean((y - mu) ** 2, axis=1, keepdims=True)
        y = (y - mu) * jax.lax.rsqrt(var + 1e-5) * gn_ref[...] + bn_ref[...]
    o_ref[...] = y


def _distill_kernel(xp_ref, w_ref, cb_ref, bng_ref, bnb_ref, o_ref, *, L):
    y = (jnp.dot(xp_ref[0:L, :], w_ref[0])
         + jnp.dot(xp_ref[1:L + 1, :], w_ref[1])
         + jnp.dot(xp_ref[2:L + 2, :], w_ref[2])
         + cb_ref[...])
    mu = jnp.mean(y, axis=0, keepdims=True)
    var = jnp.mean((y - mu) ** 2, axis=0, keepdims=True)
    y = (y - mu) * jax.lax.rsqrt(var + 1e-5) * bng_ref[...] + bnb_ref[...]
    y = jnp.where(y > 0.0, y, jnp.exp(y) - 1.0)
    ninf = jnp.full((1, y.shape[1]), -jnp.inf, jnp.float32)
    ym1 = jnp.concatenate([ninf, y[:L - 1]], axis=0)
    yp1 = jnp.concatenate([y[1:], ninf], axis=0)
    o_ref[...] = jnp.maximum(jnp.maximum(ym1, y), yp1)


def _attn_layer(x2, p, cnt, U):
    L = x2.shape[0]
    f32 = jnp.float32

    def _wT3(w):
        return w.T.reshape(HID, HEADS, DH).transpose(1, 0, 2)

    def _b3(b):
        return b.reshape(HEADS, 1, DH)

    q, kt, v = pl.pallas_call(
        _qkv_kernel,
        grid=(HEADS,),
        in_specs=[
            pl.BlockSpec((L, HID), lambda h: (0, 0)),
            pl.BlockSpec((1, HID, DH), lambda h: (h, 0, 0)),
            pl.BlockSpec((1, HID, DH), lambda h: (h, 0, 0)),
            pl.BlockSpec((1, HID, DH), lambda h: (h, 0, 0)),
            pl.BlockSpec((1, 1, DH), lambda h: (h, 0, 0)),
            pl.BlockSpec((1, 1, DH), lambda h: (h, 0, 0)),
            pl.BlockSpec((1, 1, DH), lambda h: (h, 0, 0)),
        ],
        out_specs=[
            pl.BlockSpec((1, L, DH), lambda h: (h, 0, 0)),
            pl.BlockSpec((1, DH, L), lambda h: (h, 0, 0)),
            pl.BlockSpec((1, L, DH), lambda h: (h, 0, 0)),
        ],
        out_shape=[
            jax.ShapeDtypeStruct((HEADS, L, DH), f32),
            jax.ShapeDtypeStruct((HEADS, DH, L), f32),
            jax.ShapeDtypeStruct((HEADS, L, DH), f32),
        ],
    )(x2, _wT3(p["Wq"]), _wT3(p["Wk"]), _wT3(p["Wv"]),
      _b3(p["bq"]), _b3(p["bk"]), _b3(p["bv"]))

    BQ = 512
    nqb = L // BQ
    m = pl.pallas_call(
        functools.partial(_m_kernel, LK=L),
        grid=(nqb, HEADS),
        in_specs=[
            pl.BlockSpec((1, BQ, DH), lambda qb, h: (h, qb, 0)),
            pl.BlockSpec((1, DH, L), lambda qb, h: (h, 0, 0)),
            pl.BlockSpec((BQ, L), lambda qb, h: (qb, 0)),
        ],
        out_specs=pl.BlockSpec((1, 1, 1, BQ), lambda qb, h: (h, qb, 0, 0)),
        out_shape=jax.ShapeDtypeStruct((HEADS, nqb, 1, BQ), f32),
    )(q, kt, cnt)
    m = m.reshape(HEADS, L)

    top = pl.pallas_call(
        functools.partial(_topk_kernel, U=U),
        out_shape=jax.ShapeDtypeStruct((HEADS, U), jnp.int32),
    )(m)
    topc = top.reshape(HEADS, U, 1)
    topr = top.reshape(HEADS, 1, U)

    woT3 = p["Wo"].T.reshape(HEADS, DH, HID)
    out1 = pl.pallas_call(
        functools.partial(_tail_kernel, U=U, scale=1.0 / math.sqrt(DH),
                          NH=HEADS),
        grid=(HEADS,),
        in_specs=[
            pl.BlockSpec((1, L, DH), lambda h: (h, 0, 0)),
            pl.BlockSpec((1, DH, L), lambda h: (h, 0, 0)),
            pl.BlockSpec((1, L, DH), lambda h: (h, 0, 0)),
            pl.BlockSpec((1, U, 1), lambda h: (h, 0, 0)),
            pl.BlockSpec((1, 1, U), lambda h: (h, 0, 0)),
            pl.BlockSpec((1, DH, HID), lambda h: (h, 0, 0)),
            pl.BlockSpec((1, HID), lambda h: (0, 0)),
            pl.BlockSpec((L, HID), lambda h: (0, 0)),
            pl.BlockSpec((1, HID), lambda h: (0, 0)),
            pl.BlockSpec((1, HID), lambda h: (0, 0)),
        ],
        out_specs=pl.BlockSpec((L, HID), lambda h: (0, 0)),
        out_shape=jax.ShapeDtypeStruct((L, HID), f32),
    )(q, kt, v, topc, topr, woT3, p["bo"].reshape(1, HID), x2,
      p["g1"].reshape(1, HID), p["be1"].reshape(1, HID))
    return out1


def _ffn(x2, p, final, gn, bn):
    L = x2.shape[0]
    return pl.pallas_call(
        functools.partial(_ffn_kernel, final=final),
        out_shape=jax.ShapeDtypeStruct((L, HID), jnp.float32),
    )(x2, p["W1"].T, p["b1"].reshape(1, INTER), p["W2"].T,
      p["b2"].reshape(1, HID), p["g2"].reshape(1, HID),
      p["be2"].reshape(1, HID), gn.reshape(1, HID), bn.reshape(1, HID))


def _distill(x2, p):
    L = x2.shape[0]
    xp = jnp.concatenate([x2[-1:], x2, x2[:1]], axis=0)
    wT = jnp.transpose(p["convW"], (2, 1, 0))
    b = pl.pallas_call(
        functools.partial(_distill_kernel, L=L),
        out_shape=jax.ShapeDtypeStruct((L, HID), jnp.float32),
    )(xp, wT, p["convb"].reshape(1, HID), p["bng"].reshape(1, HID),
      p["bnb"].reshape(1, HID))
    return b[::2]


def kernel(x, params):
    x2 = x[0]
    x2 = _attn_layer(x2, params["layer0"], _CNT0, _U0)
    x2 = _ffn(x2, params["layer0"], False, params["gN"], params["bN"])
    x2 = _distill(x2, params["distill"])
    x2 = _attn_layer(x2, params["layer1"], _CNT1, _U1)
    x2 = _ffn(x2, params["layer1"], True, params["gN"], params["bN"])
    return x2[None]
